# Initial kernel scaffold; baseline (speedup 1.0000x reference)
import functools

import jax
import jax.numpy as jnp
from jax import lax
from jax.experimental import pallas as pl
from jax.experimental.pallas import tpu as pltpu
from jax.experimental.pallas import tpu_sc as plsc

N = 1048576
LODS = ((32, 0), (64, 32768), (128, 294912), (256, 2392064))
N_ROWS = 19169280

NW = 32
PW = N // NW
C = 512
NCH = PW // C
VPC = C // 16


def _sc_body(x_hbm, tab_hbm, out_hbm,
             xbuf, i0, i1, i2, i3, f0, f1, f2, f3,
             r0, r1, r2, r3, obuf, s0, s1, s2, s3):
    wid = lax.axis_index("s") * 2 + lax.axis_index("c")
    idxb = (i0, i1, i2, i3)
    fracb = (f0, f1, f2, f3)
    rowsb = (r0, r1, r2, r3)
    sems = (s0, s1, s2, s3)
    lanes = jnp.arange(16, dtype=jnp.int32)
    col0 = jnp.zeros((16,), jnp.int32)
    col1 = jnp.ones((16,), jnp.int32)

    def chunk_body(k, carry):
        base = wid * PW + k * C
        pltpu.sync_copy(x_hbm.at[pl.ds(base * 3, C * 3)], xbuf)

        def gen(i, c2):
            l0 = i * 16
            p = (lanes + l0) * 3
            x = plsc.load_gather(xbuf, [p])
            y = plsc.load_gather(xbuf, [p + 1])
            z = plsc.load_gather(xbuf, [p + 2])
            for (R, off), ib, fb in zip(LODS, idxb, fracb):
                px = (x * 0.5 + 0.5) * (R - 1)
                py = (y * 0.5 + 0.5) * (R - 1)
                pz = (z * 0.5 + 0.5) * (R - 1)
                x0 = jnp.minimum(px.astype(jnp.int32), R - 2)
                y0 = jnp.minimum(py.astype(jnp.int32), R - 2)
                z0 = jnp.minimum(pz.astype(jnp.int32), R - 2)
                fb[pl.ds(l0, 16)] = px - x0.astype(jnp.float32)
                fb[pl.ds(C + l0, 16)] = py - y0.astype(jnp.float32)
                fb[pl.ds(2 * C + l0, 16)] = pz - z0.astype(jnp.float32)
                b = (x0 * R + y0) * R + z0 + off
                ib[pl.ds(0 * C + l0, 16)] = b
                ib[pl.ds(1 * C + l0, 16)] = b + 1
                ib[pl.ds(2 * C + l0, 16)] = b + R
                ib[pl.ds(3 * C + l0, 16)] = b + (R + 1)
                ib[pl.ds(4 * C + l0, 16)] = b + R * R
                ib[pl.ds(5 * C + l0, 16)] = b + (R * R + 1)
                ib[pl.ds(6 * C + l0, 16)] = b + (R * R + R)
                ib[pl.ds(7 * C + l0, 16)] = b + (R * R + R + 1)
            return c2

        lax.fori_loop(0, VPC, gen, 0)

        copies = [pltpu.async_copy(tab_hbm.at[idxb[l]], rowsb[l], sems[l])
                  for l in range(4)]

        for l in range(4):
            copies[l].wait()
            fb, rb = fracb[l], rowsb[l]

            def comb(i, c2, l=l, fb=fb, rb=rb):
                l0 = i * 16
                lane = lanes + l0
                fx = fb[pl.ds(l0, 16)]
                fy = fb[pl.ds(C + l0, 16)]
                fz = fb[pl.ds(2 * C + l0, 16)]
                v = []
                for c in range(8):
                    r = lane + c * C
                    v.append((plsc.load_gather(rb, [r, col0]),
                              plsc.load_gather(rb, [r, col1])))
                for f in range(2):
                    c00 = v[0][f] + fz * (v[1][f] - v[0][f])
                    c01 = v[2][f] + fz * (v[3][f] - v[2][f])
                    c10 = v[4][f] + fz * (v[5][f] - v[4][f])
                    c11 = v[6][f] + fz * (v[7][f] - v[6][f])
                    c0 = c00 + fy * (c01 - c00)
                    c1 = c10 + fy * (c11 - c10)
                    val = c0 + fx * (c1 - c0)
                    plsc.store_scatter(obuf, [lane * 8 + (2 * l + f)], val)
                return c2

            lax.fori_loop(0, VPC, comb, 0)

        pltpu.sync_copy(obuf, out_hbm.at[pl.ds(base * 8, C * 8)])
        return carry

    lax.fori_loop(0, NCH, chunk_body, 0)


_sc_forward = functools.partial(
    pl.kernel,
    mesh=plsc.VectorSubcoreMesh(core_axis_name="c", subcore_axis_name="s"),
    out_type=jax.ShapeDtypeStruct((N * 8,), jnp.float32),
    scratch_types=[
        pltpu.VMEM((C * 3,), jnp.float32),
        pltpu.VMEM((8 * C,), jnp.int32),
        pltpu.VMEM((8 * C,), jnp.int32),
        pltpu.VMEM((8 * C,), jnp.int32),
        pltpu.VMEM((8 * C,), jnp.int32),
        pltpu.VMEM((3 * C,), jnp.float32),
        pltpu.VMEM((3 * C,), jnp.float32),
        pltpu.VMEM((3 * C,), jnp.float32),
        pltpu.VMEM((3 * C,), jnp.float32),
        pltpu.VMEM((8 * C, 2), jnp.float32),
        pltpu.VMEM((8 * C, 2), jnp.float32),
        pltpu.VMEM((8 * C, 2), jnp.float32),
        pltpu.VMEM((8 * C, 2), jnp.float32),
        pltpu.VMEM((8 * C,), jnp.float32),
        pltpu.SemaphoreType.DMA,
        pltpu.SemaphoreType.DMA,
        pltpu.SemaphoreType.DMA,
        pltpu.SemaphoreType.DMA,
    ],
)(_sc_forward_body := None) if False else None


def _make_sc_forward():
    return functools.partial(
        pl.kernel,
        mesh=plsc.VectorSubcoreMesh(core_axis_name="c", subcore_axis_name="s"),
        out_type=jax.ShapeDtypeStruct((N * 8,), jnp.float32),
        scratch_types=[
            pltpu.VMEM((C * 3,), jnp.float32),
            pltpu.VMEM((8 * C,), jnp.int32),
            pltpu.VMEM((8 * C,), jnp.int32),
            pltpu.VMEM((8 * C,), jnp.int32),
            pltpu.VMEM((8 * C,), jnp.int32),
            pltpu.VMEM((3 * C,), jnp.float32),
            pltpu.VMEM((3 * C,), jnp.float32),
            pltpu.VMEM((3 * C,), jnp.float32),
            pltpu.VMEM((3 * C,), jnp.float32),
            pltpu.VMEM((8 * C, 2), jnp.float32),
            pltpu.VMEM((8 * C, 2), jnp.float32),
            pltpu.VMEM((8 * C, 2), jnp.float32),
            pltpu.VMEM((8 * C, 2), jnp.float32),
            pltpu.VMEM((8 * C,), jnp.float32),
            pltpu.SemaphoreType.DMA,
            pltpu.SemaphoreType.DMA,
            pltpu.SemaphoreType.DMA,
            pltpu.SemaphoreType.DMA,
        ],
    )(_sc_body)


_sc_forward = _make_sc_forward()


def kernel(input, flattened_params):
    x_flat = input.reshape(N * 3)
    tab = flattened_params.reshape(N_ROWS, 2)
    out = _sc_forward(x_flat, tab)
    return out.reshape(N, 8)


# speedup vs baseline: 7.1741x; 7.1741x over previous
import functools

import jax
import jax.numpy as jnp
from jax import lax
from jax.experimental import pallas as pl
from jax.experimental.pallas import tpu as pltpu
from jax.experimental.pallas import tpu_sc as plsc

N = 1048576
LODS = ((32, 0), (64, 32768), (128, 294912), (256, 2392064))
N_ROWS = 19169280

NW = 32
PW = N // NW
C = 256
NCH = PW // C
VPC = C // 16


def _sc_body(x_hbm, tab_hbm, out_hbm,
             xbuf, i0, i1, i2, i3, f0, f1, f2, f3,
             r0, r1, r2, r3, obuf, s0, s1, s2, s3):
    wid = lax.axis_index("s") * 2 + lax.axis_index("c")
    idxb = (i0, i1, i2, i3)
    fracb = (f0, f1, f2, f3)
    rowsb = (r0, r1, r2, r3)
    sems = (s0, s1, s2, s3)
    lanes = jnp.arange(16, dtype=jnp.int32)
    col0 = jnp.zeros((16,), jnp.int32)
    col1 = jnp.ones((16,), jnp.int32)

    def chunk_body(k, carry):
        base = wid * PW + k * C
        pltpu.sync_copy(x_hbm.at[pl.ds(base * 3, C * 3)], xbuf)

        def gen(i, c2):
            l0 = i * 16
            p = (lanes + l0) * 3
            x = plsc.load_gather(xbuf, [p])
            y = plsc.load_gather(xbuf, [p + 1])
            z = plsc.load_gather(xbuf, [p + 2])
            for (R, off), ib, fb in zip(LODS, idxb, fracb):
                px = (x * 0.5 + 0.5) * (R - 1)
                py = (y * 0.5 + 0.5) * (R - 1)
                pz = (z * 0.5 + 0.5) * (R - 1)
                x0 = jnp.minimum(px.astype(jnp.int32), R - 2)
                y0 = jnp.minimum(py.astype(jnp.int32), R - 2)
                z0 = jnp.minimum(pz.astype(jnp.int32), R - 2)
                fb[pl.ds(l0, 16)] = px - x0.astype(jnp.float32)
                fb[pl.ds(C + l0, 16)] = py - y0.astype(jnp.float32)
                fb[pl.ds(2 * C + l0, 16)] = pz - z0.astype(jnp.float32)
                b = (x0 * R + y0) * R + z0 + off
                ib[pl.ds(0 * C + l0, 16)] = b
                ib[pl.ds(1 * C + l0, 16)] = b + 1
                ib[pl.ds(2 * C + l0, 16)] = b + R
                ib[pl.ds(3 * C + l0, 16)] = b + (R + 1)
                ib[pl.ds(4 * C + l0, 16)] = b + R * R
                ib[pl.ds(5 * C + l0, 16)] = b + (R * R + 1)
                ib[pl.ds(6 * C + l0, 16)] = b + (R * R + R)
                ib[pl.ds(7 * C + l0, 16)] = b + (R * R + R + 1)
            return c2

        lax.fori_loop(0, VPC, gen, 0)

        copies = [pltpu.async_copy(tab_hbm.at[idxb[l]], rowsb[l], sems[l])
                  for l in range(4)]

        for l in range(4):
            copies[l].wait()
            fb, rb = fracb[l], rowsb[l]

            def comb(i, c2, l=l, fb=fb, rb=rb):
                l0 = i * 16
                lane = lanes + l0
                fx = fb[pl.ds(l0, 16)]
                fy = fb[pl.ds(C + l0, 16)]
                fz = fb[pl.ds(2 * C + l0, 16)]
                v = []
                for c in range(8):
                    r = lane + c * C
                    v.append((plsc.load_gather(rb, [r, col0]),
                              plsc.load_gather(rb, [r, col1])))
                for f in range(2):
                    c00 = v[0][f] + fz * (v[1][f] - v[0][f])
                    c01 = v[2][f] + fz * (v[3][f] - v[2][f])
                    c10 = v[4][f] + fz * (v[5][f] - v[4][f])
                    c11 = v[6][f] + fz * (v[7][f] - v[6][f])
                    c0 = c00 + fy * (c01 - c00)
                    c1 = c10 + fy * (c11 - c10)
                    val = c0 + fx * (c1 - c0)
                    plsc.store_scatter(obuf, [lane * 8 + (2 * l + f)], val)
                return c2

            lax.fori_loop(0, VPC, comb, 0)

        pltpu.sync_copy(obuf, out_hbm.at[pl.ds(base * 8, C * 8)])
        return carry

    lax.fori_loop(0, NCH, chunk_body, 0)


@functools.cache
def _make_sc_forward():
    return functools.partial(
        pl.kernel,
        mesh=plsc.VectorSubcoreMesh(core_axis_name="c", subcore_axis_name="s"),
        out_type=jax.ShapeDtypeStruct((N * 8,), jnp.float32),
        compiler_params=pltpu.CompilerParams(
            needs_layout_passes=False, use_tc_tiling_on_sc=False),
        scratch_types=[
            pltpu.VMEM((C * 3,), jnp.float32),
            pltpu.VMEM((8 * C,), jnp.int32),
            pltpu.VMEM((8 * C,), jnp.int32),
            pltpu.VMEM((8 * C,), jnp.int32),
            pltpu.VMEM((8 * C,), jnp.int32),
            pltpu.VMEM((3 * C,), jnp.float32),
            pltpu.VMEM((3 * C,), jnp.float32),
            pltpu.VMEM((3 * C,), jnp.float32),
            pltpu.VMEM((3 * C,), jnp.float32),
            pltpu.VMEM((8 * C, 2), jnp.float32),
            pltpu.VMEM((8 * C, 2), jnp.float32),
            pltpu.VMEM((8 * C, 2), jnp.float32),
            pltpu.VMEM((8 * C, 2), jnp.float32),
            pltpu.VMEM((8 * C,), jnp.float32),
            pltpu.SemaphoreType.DMA,
            pltpu.SemaphoreType.DMA,
            pltpu.SemaphoreType.DMA,
            pltpu.SemaphoreType.DMA,
        ],
    )(_sc_body)


def kernel(input, flattened_params):
    x_flat = input.reshape(N * 3)
    tab = flattened_params.reshape(N_ROWS, 2)
    out = _make_sc_forward()(x_flat, tab)
    return out.reshape(N, 8)


# speedup vs baseline: 48.1811x; 6.7159x over previous
import functools

import jax
import jax.numpy as jnp
from jax import lax
from jax.experimental import pallas as pl
from jax.experimental.pallas import tpu as pltpu
from jax.experimental.pallas import tpu_sc as plsc

N = 1048576
LODS = ((32, 0), (64, 32768), (128, 294912), (256, 2392064))
N_V8 = 4792320

NW = 32
PW = N // NW
C = 256
NCH = PW // C
VPC = C // 16


def _sc_body(x_hbm, tab_hbm, out_hbm,
             xbuf, i0, i1, i2, i3, f0, f1, f2, f3,
             r0, r1, r2, r3, obuf, s0, s1, s2, s3):
    wid = lax.axis_index("s") * 2 + lax.axis_index("c")
    idxb = (i0, i1, i2, i3)
    fracb = (f0, f1, f2, f3)
    rowsb = (r0, r1, r2, r3)
    sems = (s0, s1, s2, s3)
    lanes = jnp.arange(16, dtype=jnp.int32)

    def chunk_body(k, carry):
        base = wid * PW + k * C
        pltpu.sync_copy(x_hbm.at[pl.ds(base, C)], xbuf)

        def gen(i, c2):
            l0 = i * 16
            p = lanes + l0
            x = plsc.load_gather(xbuf, [p, jnp.zeros((16,), jnp.int32)])
            y = plsc.load_gather(xbuf, [p, jnp.ones((16,), jnp.int32)])
            z = plsc.load_gather(xbuf, [p, jnp.full((16,), 2, jnp.int32)])
            for (R, off), ib, fb in zip(LODS, idxb, fracb):
                px = (x * 0.5 + 0.5) * (R - 1)
                py = (y * 0.5 + 0.5) * (R - 1)
                pz = (z * 0.5 + 0.5) * (R - 1)
                x0 = jnp.minimum(px.astype(jnp.int32), R - 2)
                y0 = jnp.minimum(py.astype(jnp.int32), R - 2)
                z0 = jnp.minimum(pz.astype(jnp.int32), R - 2)
                fb[pl.ds(l0, 16)] = px - x0.astype(jnp.float32)
                fb[pl.ds(C + l0, 16)] = py - y0.astype(jnp.float32)
                fb[pl.ds(2 * C + l0, 16)] = pz - z0.astype(jnp.float32)
                b = (x0 * R + y0) * R + z0 + off
                fb[pl.ds(3 * C + l0, 16)] = (
                    jnp.bitwise_and(z0, 3).astype(jnp.float32))
                ib[pl.ds(0 * C + l0, 16)] = jnp.right_shift(b, 2)
                ib[pl.ds(1 * C + l0, 16)] = jnp.right_shift(b + 1, 2)
                ib[pl.ds(2 * C + l0, 16)] = jnp.right_shift(b + R, 2)
                ib[pl.ds(3 * C + l0, 16)] = jnp.right_shift(b + (R + 1), 2)
                ib[pl.ds(4 * C + l0, 16)] = jnp.right_shift(b + R * R, 2)
                ib[pl.ds(5 * C + l0, 16)] = jnp.right_shift(b + (R * R + 1), 2)
                ib[pl.ds(6 * C + l0, 16)] = jnp.right_shift(b + (R * R + R), 2)
                ib[pl.ds(7 * C + l0, 16)] = jnp.right_shift(
                    b + (R * R + R + 1), 2)
            return c2

        lax.fori_loop(0, VPC, gen, 0)

        copies = [pltpu.async_copy(tab_hbm.at[idxb[l]], rowsb[l], sems[l])
                  for l in range(4)]

        for l in range(4):
            copies[l].wait()
            fb, rb = fracb[l], rowsb[l]

            def comb(i, c2, l=l, fb=fb, rb=rb):
                l0 = i * 16
                lane = lanes + l0
                fx = fb[pl.ds(l0, 16)]
                fy = fb[pl.ds(C + l0, 16)]
                fz = fb[pl.ds(2 * C + l0, 16)]
                zlow = fb[pl.ds(3 * C + l0, 16)].astype(jnp.int32)
                czero = zlow * 2
                cone = jnp.bitwise_and(zlow + 1, 3) * 2
                v = []
                for c in range(8):
                    r = lane + c * C
                    col = cone if (c & 1) else czero
                    v.append((plsc.load_gather(rb, [r, col]),
                              plsc.load_gather(rb, [r, col + 1])))
                for f in range(2):
                    c00 = v[0][f] + fz * (v[1][f] - v[0][f])
                    c01 = v[2][f] + fz * (v[3][f] - v[2][f])
                    c10 = v[4][f] + fz * (v[5][f] - v[4][f])
                    c11 = v[6][f] + fz * (v[7][f] - v[6][f])
                    c0 = c00 + fy * (c01 - c00)
                    c1 = c10 + fy * (c11 - c10)
                    val = c0 + fx * (c1 - c0)
                    plsc.store_scatter(
                        obuf, [lane, jnp.full((16,), 2 * l + f, jnp.int32)],
                        val)
                return c2

            lax.fori_loop(0, VPC, comb, 0)

        pltpu.sync_copy(obuf, out_hbm.at[pl.ds(base, C)])
        return carry

    lax.fori_loop(0, NCH, chunk_body, 0)


@functools.cache
def _make_sc_forward():
    return functools.partial(
        pl.kernel,
        mesh=plsc.VectorSubcoreMesh(core_axis_name="c", subcore_axis_name="s"),
        out_type=jax.ShapeDtypeStruct((N, 8), jnp.float32),
        compiler_params=pltpu.CompilerParams(
            needs_layout_passes=False, use_tc_tiling_on_sc=False),
        scratch_types=[
            pltpu.VMEM((C, 3), jnp.float32),
            pltpu.VMEM((8 * C,), jnp.int32),
            pltpu.VMEM((8 * C,), jnp.int32),
            pltpu.VMEM((8 * C,), jnp.int32),
            pltpu.VMEM((8 * C,), jnp.int32),
            pltpu.VMEM((4 * C,), jnp.float32),
            pltpu.VMEM((4 * C,), jnp.float32),
            pltpu.VMEM((4 * C,), jnp.float32),
            pltpu.VMEM((4 * C,), jnp.float32),
            pltpu.VMEM((8 * C, 8), jnp.float32),
            pltpu.VMEM((8 * C, 8), jnp.float32),
            pltpu.VMEM((8 * C, 8), jnp.float32),
            pltpu.VMEM((8 * C, 8), jnp.float32),
            pltpu.VMEM((C, 8), jnp.float32),
            pltpu.SemaphoreType.DMA,
            pltpu.SemaphoreType.DMA,
            pltpu.SemaphoreType.DMA,
            pltpu.SemaphoreType.DMA,
        ],
    )(_sc_body)


def kernel(input, flattened_params):
    tab = flattened_params.reshape(N_V8, 8)
    return _make_sc_forward()(input, tab)
